# baseline (speedup 1.0000x reference)
import jax
import jax.numpy as jnp
from jax import lax
from jax.experimental import pallas as pl
from jax.experimental.pallas import tpu as pltpu
from jax.experimental.pallas import tpu_sc as plsc

BATCH = 16384
EMB = 32
NUM_CORES = 2
NUM_SUBCORES = 16
NUM_WORKERS = NUM_CORES * NUM_SUBCORES
BPW = BATCH // NUM_WORKERS
CHUNK = 16
NCH = BPW // CHUNK
LANE = 128
NGRP = 3


def _emb_body(uidx_hbm, iidx_hbm, utab_hbm, itab_hbm, uout_hbm, iout_hbm,
              idx_vm, win_v, out_v, sems):
    w = lax.axis_index("s") * NUM_CORES + lax.axis_index("c")
    base = pl.multiple_of(w * BPW, LANE)

    rows0 = jnp.arange(16, dtype=jnp.int32)
    rows1 = rows0 + 16

    def load_vec(c):
        return idx_vm[pl.ds(pl.multiple_of(c * CHUNK, CHUNK), CHUNK)]

    def split(vec):
        return [jnp.squeeze(lax.slice(vec, (k,), (k + 1,)))
                for k in range(CHUNK)]

    def run_table(idx_hbm, tab_hbm, out_hbm):
        pltpu.sync_copy(idx_hbm.at[pl.ds(base, BPW)], idx_vm)

        def fire_half(scalars, half, g):
            for k in range(8):
                r = scalars[half * 8 + k]
                start = pl.multiple_of(r - (r & (LANE - 1)), LANE)
                pltpu.async_copy(
                    tab_hbm.at[:, pl.ds(start, LANE)],
                    win_v.at[g, k],
                    sems.at[g * 8 + k],
                )

        def extract_half(scalars, cprev, half, g):
            for k in range(8):
                pltpu.make_async_copy(
                    tab_hbm.at[:, pl.ds(0, LANE)],
                    win_v.at[g, k],
                    sems.at[g * 8 + k],
                ).wait()
                r = scalars[half * 8 + k]
                bl = jnp.full((16,), r & (LANE - 1), jnp.int32)
                cj = jnp.full((16,), cprev * CHUNK + half * 8 + k, jnp.int32)
                v0 = plsc.load_gather(win_v.at[g, k], [rows0, bl])
                v1 = plsc.load_gather(win_v.at[g, k], [rows1, bl])
                plsc.store_scatter(out_v, [rows0, cj], v0)
                plsc.store_scatter(out_v, [rows1, cj], v1)

        vec0 = load_vec(0)
        s0 = split(vec0)
        fire_half(s0, 0, 0)
        fire_half(s0, 1, 1)

        def step(c, vec_prev):
            vec = load_vec(c)
            s = split(vec)
            sp = split(vec_prev)
            ga = lax.rem(2 * c, NGRP)
            gb = lax.rem(2 * c + 1, NGRP)
            gap = lax.rem(2 * c - 2, NGRP)
            gbp = lax.rem(2 * c - 1, NGRP)
            fire_half(s, 0, ga)
            extract_half(sp, c - 1, 0, gap)
            fire_half(s, 1, gb)
            extract_half(sp, c - 1, 1, gbp)
            return vec

        vec_last = lax.fori_loop(1, NCH, step, vec0)
        sl = split(vec_last)
        extract_half(sl, NCH - 1, 0, lax.rem(2 * (NCH - 1), NGRP))
        extract_half(sl, NCH - 1, 1, lax.rem(2 * (NCH - 1) + 1, NGRP))
        pltpu.sync_copy(out_v, out_hbm.at[:, pl.ds(base, BPW)])

    run_table(uidx_hbm, utab_hbm, uout_hbm)
    run_table(iidx_hbm, itab_hbm, iout_hbm)


@jax.jit
def _emb_lookup(uidx, iidx, utab_t, itab_t):
    mesh = plsc.VectorSubcoreMesh(
        core_axis_name="c", subcore_axis_name="s",
        num_cores=NUM_CORES, num_subcores=NUM_SUBCORES)
    return pl.kernel(
        _emb_body,
        out_type=[
            jax.ShapeDtypeStruct((EMB, BATCH), jnp.float32),
            jax.ShapeDtypeStruct((EMB, BATCH), jnp.float32),
        ],
        mesh=mesh,
        compiler_params=pltpu.CompilerParams(needs_layout_passes=False),
        scratch_types=[
            pltpu.VMEM((BPW,), jnp.int32),
            pltpu.VMEM((NGRP, 8, EMB, LANE), jnp.float32),
            pltpu.VMEM((EMB, BPW), jnp.float32),
            pltpu.SemaphoreType.DMA((NGRP * 8,)),
        ],
    )(uidx, iidx, utab_t, itab_t)


def kernel(user_item_pairs, user_embeddings, item_embeddings):
    uidx = user_item_pairs[:, 0].astype(jnp.int32)
    iidx = user_item_pairs[:, 1].astype(jnp.int32)
    uout_t, iout_t = _emb_lookup(
        uidx, iidx, user_embeddings.T, item_embeddings.T)
    return (uout_t.T, iout_t.T)
